# Initial kernel scaffold; baseline (speedup 1.0000x reference)
import jax
import jax.numpy as jnp
from jax.experimental import pallas as pl


def kernel(x, edge_index, W1, b1, W_mu, b_mu, W_ls, b_ls):
    raise NotImplementedError("write your pallas kernel here")


# speedup vs baseline: 26.6051x; 26.6051x over previous
import functools

import jax
import jax.numpy as jnp
from jax import lax
from jax.experimental import pallas as pl
from jax.experimental.pallas import tpu as pltpu
from jax.experimental.pallas import tpu_sc as plsc

N = 10000
E = 320000
C = 128
NC = 2
NS = 16
K = 125
CH = E // (NC * NS * K)
EW = E // (NC * NS)
DR = 1000
ND = N // DR
ZC = 40
NP = 10240
SW = NP // ND

_mesh = plsc.VectorSubcoreMesh(
    core_axis_name="c", subcore_axis_name="s", num_cores=NC, num_subcores=NS)


@functools.partial(
    pl.kernel,
    out_type=jax.ShapeDtypeStruct((NC, NP), jnp.float32),
    mesh=_mesh,
    scratch_types=[
        pltpu.VMEM((EW,), jnp.int32),
        pltpu.VMEM((NP,), jnp.float32),
        pltpu.VMEM((SW,), jnp.float32),
        pltpu.VMEM((SW,), jnp.float32),
        pltpu.VMEM_SHARED((NS, NP), jnp.float32),
        pltpu.SemaphoreType.DMA,
    ],
    compiler_params=pltpu.CompilerParams(needs_layout_passes=False),
)
def _deg_kernel(dst_hbm, out_hbm, dst_v, cnt_v, tmp_v, sum_v, part_sh, sem):
    ci = lax.axis_index("c")
    si = lax.axis_index("s")
    zeros16 = jnp.zeros((16,), jnp.float32)
    ones16 = jnp.ones((16,), jnp.float32)

    def z1(i, _):
        cnt_v[pl.ds(16 * i, 16)] = zeros16
        return 0

    lax.fori_loop(0, NP // 16, z1, 0)

    pltpu.sync_copy(dst_hbm.at[ci, si], dst_v)

    def cnt(t, _):
        d = dst_v[pl.ds(16 * t, 16)]
        plsc.addupdate_scatter(cnt_v, [d], ones16)
        return 0

    lax.fori_loop(0, EW // 16, cnt, 0)

    pltpu.sync_copy(cnt_v, part_sh.at[si])
    plsc.subcore_barrier()

    @pl.when(si < ND)
    def _():
        pltpu.sync_copy(part_sh.at[0, pl.ds(SW * si, SW)], sum_v)
        for k in range(1, NS):
            pltpu.sync_copy(part_sh.at[k, pl.ds(SW * si, SW)], tmp_v)

            def addv(i, _):
                sl = pl.ds(16 * i, 16)
                sum_v[sl] = sum_v[sl] + tmp_v[sl]
                return 0

            lax.fori_loop(0, SW // 16, addv, 0)
        pltpu.sync_copy(sum_v, out_hbm.at[ci, pl.ds(SW * si, SW)])


@functools.partial(
    pl.kernel,
    out_type=jax.ShapeDtypeStruct((NC, N, C), jnp.float32),
    mesh=_mesh,
    scratch_types=[
        pltpu.VMEM((CH, K), jnp.int32),
        pltpu.VMEM((CH, K), jnp.int32),
        pltpu.VMEM((K, C), jnp.float32),
        pltpu.VMEM((ZC, C), jnp.float32),
        pltpu.VMEM_SHARED((N, C), jnp.float32),
        pltpu.SemaphoreType.DMA,
    ],
)
def _agg_kernel(z_hbm, src_hbm, dst_hbm, out_hbm, src_v, dst_v, rows_v,
                zbuf_v, acc_sh, sem):
    ci = lax.axis_index("c")
    si = lax.axis_index("s")

    def zrow(i, _):
        for j in range(C // 16):
            zbuf_v[i, pl.ds(16 * j, 16)] = jnp.zeros((16,), jnp.float32)
        return 0

    lax.fori_loop(0, ZC, zrow, 0)

    @pl.when(si < ND)
    def _():
        for t in range(DR // ZC):
            pltpu.sync_copy(zbuf_v, acc_sh.at[pl.ds(si * DR + t * ZC, ZC)])

    plsc.subcore_barrier()

    pltpu.sync_copy(src_hbm.at[ci, si], src_v)
    pltpu.sync_copy(dst_hbm.at[ci, si], dst_v)

    def body(j, _):
        pltpu.async_copy(z_hbm.at[src_v.at[j]], rows_v, sem).wait()
        pltpu.sync_copy(rows_v, acc_sh.at[dst_v.at[j]], add=True)
        return 0

    lax.fori_loop(0, CH, body, 0)
    plsc.subcore_barrier()

    @pl.when(si < ND)
    def _():
        pltpu.sync_copy(acc_sh.at[pl.ds(si * DR, DR)],
                        out_hbm.at[ci, pl.ds(si * DR, DR)])


R = 1000


def _dinv_block(deg):
    return lax.rsqrt(deg[:] + 1.0)


def _mm_prescale_body(x, w, deg, o):
    z = jnp.dot(x[:], w[:], preferred_element_type=jnp.float32)
    o[:] = z * _dinv_block(deg)


def _layer2_body(a, z1p, b1, deg, w, o):
    dinv = _dinv_block(deg)
    h = jnp.maximum(dinv * (a[0] + a[1] + z1p[:]) + b1[:], 0.0)
    z2 = jnp.dot(h, w[:], preferred_element_type=jnp.float32)
    o[:] = z2 * dinv


def _final_body(a, z2p, bcat, deg, o):
    o[:] = _dinv_block(deg) * (a[0] + a[1] + z2p[:]) + bcat[:]


def _rows(width):
    return pl.BlockSpec((R, width), lambda i: (i, 0))


def _pair(width):
    return pl.BlockSpec((NC, R, width), lambda i: (0, i, 0))


def _full(shape):
    return pl.BlockSpec(shape, lambda i: tuple(0 for _ in shape))


def _nxc():
    return jax.ShapeDtypeStruct((N, C), jnp.float32)


def kernel(x, edge_index, W1, b1, W_mu, b_mu, W_ls, b_ls):
    src_r = edge_index[0].reshape(NC, NS, CH, K)
    dst_r = edge_index[1].reshape(NC, NS, CH, K)
    dst_flat = edge_index[1].reshape(NC, NS, EW)

    degk = _deg_kernel(dst_flat)
    deg = (degk[0] + degk[1])[:N].reshape(N, 1)

    z1p = pl.pallas_call(
        _mm_prescale_body,
        grid=(N // R,),
        in_specs=[_rows(C), _full((C, C)), _rows(1)],
        out_specs=_rows(C),
        out_shape=_nxc(),
    )(x, W1, deg)

    acc1 = _agg_kernel(z1p, src_r, dst_r)

    w_cat = jnp.concatenate([W_mu, W_ls], axis=1)
    b_cat = jnp.concatenate([b_mu, b_ls]).reshape(1, C)

    z2p = pl.pallas_call(
        _layer2_body,
        grid=(N // R,),
        in_specs=[_pair(C), _rows(C), _full((1, C)), _rows(1),
                  _full((C, C))],
        out_specs=_rows(C),
        out_shape=_nxc(),
    )(acc1, z1p, b1.reshape(1, C), deg, w_cat)

    acc2 = _agg_kernel(z2p, src_r, dst_r)

    out = pl.pallas_call(
        _final_body,
        grid=(N // R,),
        in_specs=[_pair(C), _rows(C), _full((1, C)), _rows(1)],
        out_specs=_rows(C),
        out_shape=_nxc(),
    )(acc2, z2p, b_cat, deg)

    return (out[:, :C // 2], out[:, C // 2:])


# speedup vs baseline: 31.1861x; 1.1722x over previous
import functools

import jax
import jax.numpy as jnp
from jax import lax
from jax.experimental import pallas as pl
from jax.experimental.pallas import tpu as pltpu
from jax.experimental.pallas import tpu_sc as plsc

N = 10000
E = 320000
C = 128
NC = 2
NS = 16
K = 100
CH = E // (NC * NS * K)
EW = E // (NC * NS)
DR = 1000
ND = N // DR
ZC = 40
NP = 10240
SW = NP // ND

_mesh = plsc.VectorSubcoreMesh(
    core_axis_name="c", subcore_axis_name="s", num_cores=NC, num_subcores=NS)


@functools.partial(
    pl.kernel,
    out_type=jax.ShapeDtypeStruct((NC, NP), jnp.float32),
    mesh=_mesh,
    scratch_types=[
        pltpu.VMEM((EW,), jnp.int32),
        pltpu.VMEM((NP,), jnp.float32),
        pltpu.VMEM((SW,), jnp.float32),
        pltpu.VMEM((SW,), jnp.float32),
        pltpu.VMEM_SHARED((NS, NP), jnp.float32),
        pltpu.SemaphoreType.DMA,
    ],
    compiler_params=pltpu.CompilerParams(needs_layout_passes=False),
)
def _deg_kernel(dst_hbm, out_hbm, dst_v, cnt_v, tmp_v, sum_v, part_sh, sem):
    ci = lax.axis_index("c")
    si = lax.axis_index("s")
    zeros16 = jnp.zeros((16,), jnp.float32)
    ones16 = jnp.ones((16,), jnp.float32)

    def z1(i, _):
        cnt_v[pl.ds(16 * i, 16)] = zeros16
        return 0

    lax.fori_loop(0, NP // 16, z1, 0)

    pltpu.sync_copy(dst_hbm.at[ci, si], dst_v)

    def cnt(t, _):
        d = dst_v[pl.ds(16 * t, 16)]
        plsc.addupdate_scatter(cnt_v, [d], ones16)
        return 0

    lax.fori_loop(0, EW // 16, cnt, 0)

    pltpu.sync_copy(cnt_v, part_sh.at[si])
    plsc.subcore_barrier()

    @pl.when(si < ND)
    def _():
        pltpu.sync_copy(part_sh.at[0, pl.ds(SW * si, SW)], sum_v)
        for k in range(1, NS):
            pltpu.sync_copy(part_sh.at[k, pl.ds(SW * si, SW)], tmp_v)

            def addv(i, _):
                sl = pl.ds(16 * i, 16)
                sum_v[sl] = sum_v[sl] + tmp_v[sl]
                return 0

            lax.fori_loop(0, SW // 16, addv, 0)
        pltpu.sync_copy(sum_v, out_hbm.at[ci, pl.ds(SW * si, SW)])


@functools.partial(
    pl.kernel,
    out_type=jax.ShapeDtypeStruct((NC, N, C), jnp.float32),
    mesh=_mesh,
    scratch_types=[
        pltpu.VMEM((CH // 2, K), jnp.int32),
        pltpu.VMEM((CH // 2, K), jnp.int32),
        pltpu.VMEM((K, C), jnp.float32),
        pltpu.VMEM((K, C), jnp.float32),
        pltpu.VMEM_SHARED((N, C), jnp.float32),
        pltpu.SemaphoreType.DMA,
        pltpu.SemaphoreType.DMA,
    ],
)
def _agg_kernel(z_hbm, src_hbm, dst_hbm, out_hbm, src_v, dst_v, rows0_v,
                rows1_v, acc_sh, sem_g, sem_s):
    ci = lax.axis_index("c")
    si = lax.axis_index("s")

    def zrow(i, _):
        for j in range(C // 16):
            rows0_v[i, pl.ds(16 * j, 16)] = jnp.zeros((16,), jnp.float32)
        return 0

    lax.fori_loop(0, K, zrow, 0)

    @pl.when(si < ND)
    def _():
        nfull, rem = divmod(DR, K)
        for t in range(nfull):
            pltpu.sync_copy(rows0_v, acc_sh.at[pl.ds(si * DR + t * K, K)])
        if rem:
            pltpu.sync_copy(rows0_v.at[pl.ds(0, rem)],
                            acc_sh.at[pl.ds(si * DR + nfull * K, rem)])

    plsc.subcore_barrier()

    HC = CH // 2
    for h in range(2):
        pltpu.sync_copy(src_hbm.at[ci, si, h], src_v)
        pltpu.sync_copy(dst_hbm.at[ci, si, h], dst_v)
        pltpu.async_copy(z_hbm.at[src_v.at[0]], rows0_v, sem_g)

        def body(g, _):
            for b in range(2):
                j = 2 * g + b
                cur = rows0_v if b == 0 else rows1_v
                oth = rows1_v if b == 0 else rows0_v
                pltpu.make_async_copy(z_hbm.at[src_v.at[j]], cur,
                                      sem_g).wait()

                @pl.when(j > 0)
                def _():
                    pltpu.make_async_copy(
                        oth, acc_sh.at[dst_v.at[j - 1]], sem_s).wait()

                @pl.when(j + 1 < HC)
                def _():
                    pltpu.async_copy(z_hbm.at[src_v.at[j + 1]], oth, sem_g)

                pltpu.async_copy(cur, acc_sh.at[dst_v.at[j]], sem_s,
                                 add=True)
            return 0

        lax.fori_loop(0, HC // 2, body, 0)
        pltpu.make_async_copy(rows1_v, acc_sh.at[dst_v.at[HC - 1]],
                              sem_s).wait()
    plsc.subcore_barrier()

    @pl.when(si < ND)
    def _():
        pltpu.sync_copy(acc_sh.at[pl.ds(si * DR, DR)],
                        out_hbm.at[ci, pl.ds(si * DR, DR)])


R = 1000


def _dinv_block(deg):
    return lax.rsqrt(deg[:] + 1.0)


def _mm_prescale_body(x, w, deg, o):
    z = jnp.dot(x[:], w[:], preferred_element_type=jnp.float32)
    o[:] = z * _dinv_block(deg)


def _layer2_body(a, z1p, b1, deg, w, o):
    dinv = _dinv_block(deg)
    h = jnp.maximum(dinv * (a[0] + a[1] + z1p[:]) + b1[:], 0.0)
    z2 = jnp.dot(h, w[:], preferred_element_type=jnp.float32)
    o[:] = z2 * dinv


def _final_body(a, z2p, bcat, deg, o):
    o[:] = _dinv_block(deg) * (a[0] + a[1] + z2p[:]) + bcat[:]


def _rows(width):
    return pl.BlockSpec((R, width), lambda i: (i, 0))


def _pair(width):
    return pl.BlockSpec((NC, R, width), lambda i: (0, i, 0))


def _full(shape):
    return pl.BlockSpec(shape, lambda i: tuple(0 for _ in shape))


def _nxc():
    return jax.ShapeDtypeStruct((N, C), jnp.float32)


def kernel(x, edge_index, W1, b1, W_mu, b_mu, W_ls, b_ls):
    src_r = edge_index[0].reshape(NC, NS, 2, CH // 2, K)
    dst_r = edge_index[1].reshape(NC, NS, 2, CH // 2, K)
    dst_flat = edge_index[1].reshape(NC, NS, EW)

    degk = _deg_kernel(dst_flat)
    deg = (degk[0] + degk[1])[:N].reshape(N, 1)

    z1p = pl.pallas_call(
        _mm_prescale_body,
        grid=(N // R,),
        in_specs=[_rows(C), _full((C, C)), _rows(1)],
        out_specs=_rows(C),
        out_shape=_nxc(),
    )(x, W1, deg)

    acc1 = _agg_kernel(z1p, src_r, dst_r)

    w_cat = jnp.concatenate([W_mu, W_ls], axis=1)
    b_cat = jnp.concatenate([b_mu, b_ls]).reshape(1, C)

    z2p = pl.pallas_call(
        _layer2_body,
        grid=(N // R,),
        in_specs=[_pair(C), _rows(C), _full((1, C)), _rows(1),
                  _full((C, C))],
        out_specs=_rows(C),
        out_shape=_nxc(),
    )(acc1, z1p, b1.reshape(1, C), deg, w_cat)

    acc2 = _agg_kernel(z2p, src_r, dst_r)

    out = pl.pallas_call(
        _final_body,
        grid=(N // R,),
        in_specs=[_pair(C), _rows(C), _full((1, C)), _rows(1)],
        out_specs=_rows(C),
        out_shape=_nxc(),
    )(acc2, z2p, b_cat, deg)

    return (out[:, :C // 2], out[:, C // 2:])


# speedup vs baseline: 33.0928x; 1.0611x over previous
import functools

import jax
import jax.numpy as jnp
from jax import lax
from jax.experimental import pallas as pl
from jax.experimental.pallas import tpu as pltpu
from jax.experimental.pallas import tpu_sc as plsc

N = 10000
E = 320000
C = 128
NC = 2
NS = 16
K = 125
CH = E // (NC * NS * K)
EW = E // (NC * NS)
DR = 1000
ND = N // DR
ZC = 40
NP = 10240
SW = NP // ND

_mesh = plsc.VectorSubcoreMesh(
    core_axis_name="c", subcore_axis_name="s", num_cores=NC, num_subcores=NS)


@functools.partial(
    pl.kernel,
    out_type=jax.ShapeDtypeStruct((NC, NP), jnp.float32),
    mesh=_mesh,
    scratch_types=[
        pltpu.VMEM((EW,), jnp.int32),
        pltpu.VMEM((NP,), jnp.float32),
        pltpu.VMEM((SW,), jnp.float32),
        pltpu.VMEM((SW,), jnp.float32),
        pltpu.VMEM_SHARED((NS, NP), jnp.float32),
        pltpu.SemaphoreType.DMA,
    ],
    compiler_params=pltpu.CompilerParams(needs_layout_passes=False),
)
def _deg_kernel(dst_hbm, out_hbm, dst_v, cnt_v, tmp_v, sum_v, part_sh, sem):
    ci = lax.axis_index("c")
    si = lax.axis_index("s")
    zeros16 = jnp.zeros((16,), jnp.float32)
    ones16 = jnp.ones((16,), jnp.float32)

    def z1(i, _):
        cnt_v[pl.ds(16 * i, 16)] = zeros16
        return 0

    lax.fori_loop(0, NP // 16, z1, 0)

    pltpu.sync_copy(dst_hbm.at[ci, si], dst_v)

    def cnt(t, _):
        d = dst_v[pl.ds(16 * t, 16)]
        plsc.addupdate_scatter(cnt_v, [d], ones16)
        return 0

    lax.fori_loop(0, EW // 16, cnt, 0)

    pltpu.sync_copy(cnt_v, part_sh.at[si])
    plsc.subcore_barrier()

    @pl.when(si < ND)
    def _():
        pltpu.sync_copy(part_sh.at[0, pl.ds(SW * si, SW)], sum_v)
        for k in range(1, NS):
            pltpu.sync_copy(part_sh.at[k, pl.ds(SW * si, SW)], tmp_v)

            def addv(i, _):
                sl = pl.ds(16 * i, 16)
                sum_v[sl] = sum_v[sl] + tmp_v[sl]
                return 0

            lax.fori_loop(0, SW // 16, addv, 0)
        pltpu.sync_copy(sum_v, out_hbm.at[ci, pl.ds(SW * si, SW)])


@functools.partial(
    pl.kernel,
    out_type=jax.ShapeDtypeStruct((NC, N, C), jnp.float32),
    mesh=_mesh,
    scratch_types=[
        pltpu.VMEM((CH // 2, K), jnp.int32),
        pltpu.VMEM((CH // 2, K), jnp.int32),
        pltpu.VMEM((K, C), jnp.float32),
        pltpu.VMEM((K, C), jnp.float32),
        pltpu.VMEM_SHARED((N, C), jnp.float32),
        pltpu.SemaphoreType.DMA,
        pltpu.SemaphoreType.DMA,
    ],
)
def _agg_kernel(z_hbm, src_hbm, dst_hbm, out_hbm, src_v, dst_v, rows0_v,
                rows1_v, acc_sh, sem_g, sem_s):
    ci = lax.axis_index("c")
    si = lax.axis_index("s")

    def zrow(i, _):
        for j in range(C // 16):
            rows0_v[i, pl.ds(16 * j, 16)] = jnp.zeros((16,), jnp.float32)
        return 0

    lax.fori_loop(0, K, zrow, 0)

    @pl.when(si < ND)
    def _():
        nfull, rem = divmod(DR, K)
        for t in range(nfull):
            pltpu.sync_copy(rows0_v, acc_sh.at[pl.ds(si * DR + t * K, K)])
        if rem:
            pltpu.sync_copy(rows0_v.at[pl.ds(0, rem)],
                            acc_sh.at[pl.ds(si * DR + nfull * K, rem)])

    plsc.subcore_barrier()

    HC = CH // 2
    for h in range(2):
        pltpu.sync_copy(src_hbm.at[ci, si, h], src_v)
        pltpu.sync_copy(dst_hbm.at[ci, si, h], dst_v)
        pltpu.async_copy(z_hbm.at[src_v.at[0]], rows0_v, sem_g)

        def body(g, _):
            for b in range(2):
                j = 2 * g + b
                cur = rows0_v if b == 0 else rows1_v
                oth = rows1_v if b == 0 else rows0_v
                pltpu.make_async_copy(z_hbm.at[src_v.at[j]], cur,
                                      sem_g).wait()
                pltpu.async_copy(cur, acc_sh.at[dst_v.at[j]], sem_s,
                                 add=True)

                @pl.when(j > 0)
                def _():
                    pltpu.make_async_copy(
                        oth, acc_sh.at[dst_v.at[j - 1]], sem_s).wait()

                @pl.when(j + 1 < HC)
                def _():
                    pltpu.async_copy(z_hbm.at[src_v.at[j + 1]], oth, sem_g)
            return 0

        lax.fori_loop(0, HC // 2, body, 0)
        pltpu.make_async_copy(rows1_v, acc_sh.at[dst_v.at[HC - 1]],
                              sem_s).wait()
    plsc.subcore_barrier()

    @pl.when(si < ND)
    def _():
        pltpu.sync_copy(acc_sh.at[pl.ds(si * DR, DR)],
                        out_hbm.at[ci, pl.ds(si * DR, DR)])


R = 1000


def _dinv_block(deg):
    return lax.rsqrt(deg[:] + 1.0)


def _mm_prescale_body(x, w, deg, o):
    z = jnp.dot(x[:], w[:], preferred_element_type=jnp.float32)
    o[:] = z * _dinv_block(deg)


def _layer2_body(a, z1p, b1, deg, w, o):
    dinv = _dinv_block(deg)
    h = jnp.maximum(dinv * (a[0] + a[1] + z1p[:]) + b1[:], 0.0)
    z2 = jnp.dot(h, w[:], preferred_element_type=jnp.float32)
    o[:] = z2 * dinv


def _final_body(a, z2p, bcat, deg, o):
    o[:] = _dinv_block(deg) * (a[0] + a[1] + z2p[:]) + bcat[:]


def _rows(width):
    return pl.BlockSpec((R, width), lambda i: (i, 0))


def _pair(width):
    return pl.BlockSpec((NC, R, width), lambda i: (0, i, 0))


def _full(shape):
    return pl.BlockSpec(shape, lambda i: tuple(0 for _ in shape))


def _nxc():
    return jax.ShapeDtypeStruct((N, C), jnp.float32)


def kernel(x, edge_index, W1, b1, W_mu, b_mu, W_ls, b_ls):
    src_r = edge_index[0].reshape(NC, NS, 2, CH // 2, K)
    dst_r = edge_index[1].reshape(NC, NS, 2, CH // 2, K)
    dst_flat = edge_index[1].reshape(NC, NS, EW)

    degk = _deg_kernel(dst_flat)
    deg = (degk[0] + degk[1])[:N].reshape(N, 1)

    z1p = pl.pallas_call(
        _mm_prescale_body,
        grid=(N // R,),
        in_specs=[_rows(C), _full((C, C)), _rows(1)],
        out_specs=_rows(C),
        out_shape=_nxc(),
    )(x, W1, deg)

    acc1 = _agg_kernel(z1p, src_r, dst_r)

    w_cat = jnp.concatenate([W_mu, W_ls], axis=1)
    b_cat = jnp.concatenate([b_mu, b_ls]).reshape(1, C)

    z2p = pl.pallas_call(
        _layer2_body,
        grid=(N // R,),
        in_specs=[_pair(C), _rows(C), _full((1, C)), _rows(1),
                  _full((C, C))],
        out_specs=_rows(C),
        out_shape=_nxc(),
    )(acc1, z1p, b1.reshape(1, C), deg, w_cat)

    acc2 = _agg_kernel(z2p, src_r, dst_r)

    out = pl.pallas_call(
        _final_body,
        grid=(N // R,),
        in_specs=[_pair(C), _rows(C), _full((1, C)), _rows(1)],
        out_specs=_rows(C),
        out_shape=_nxc(),
    )(acc2, z2p, b_cat, deg)

    return (out[:, :C // 2], out[:, C // 2:])


# speedup vs baseline: 34.3108x; 1.0368x over previous
import functools

import jax
import jax.numpy as jnp
from jax import lax
from jax.experimental import pallas as pl
from jax.experimental.pallas import tpu as pltpu
from jax.experimental.pallas import tpu_sc as plsc

N = 10000
E = 320000
C = 128
NC = 2
NS = 16
K = 125
CH = E // (NC * NS * K)
EW = E // (NC * NS)
DR = 1000
ND = N // DR
ZC = 40
NP = 10240
SW = NP // ND

_mesh = plsc.VectorSubcoreMesh(
    core_axis_name="c", subcore_axis_name="s", num_cores=NC, num_subcores=NS)


@functools.partial(
    pl.kernel,
    out_type=jax.ShapeDtypeStruct((NC, NP), jnp.float32),
    mesh=_mesh,
    scratch_types=[
        pltpu.VMEM((EW,), jnp.int32),
        pltpu.VMEM((NP,), jnp.float32),
        pltpu.VMEM((SW,), jnp.float32),
        pltpu.VMEM((SW,), jnp.float32),
        pltpu.VMEM_SHARED((NS, NP), jnp.float32),
        pltpu.SemaphoreType.DMA,
    ],
    compiler_params=pltpu.CompilerParams(needs_layout_passes=False),
)
def _deg_kernel(dst_hbm, out_hbm, dst_v, cnt_v, tmp_v, sum_v, part_sh, sem):
    ci = lax.axis_index("c")
    si = lax.axis_index("s")
    zeros16 = jnp.zeros((16,), jnp.float32)
    ones16 = jnp.ones((16,), jnp.float32)

    def z1(i, _):
        cnt_v[pl.ds(16 * i, 16)] = zeros16
        return 0

    lax.fori_loop(0, NP // 16, z1, 0)

    pltpu.sync_copy(dst_hbm.at[ci, si], dst_v)

    def cnt(t, _):
        d = dst_v[pl.ds(16 * t, 16)]
        plsc.addupdate_scatter(cnt_v, [d], ones16)
        return 0

    lax.fori_loop(0, EW // 16, cnt, 0)

    pltpu.sync_copy(cnt_v, part_sh.at[si])
    plsc.subcore_barrier()

    @pl.when(si < ND)
    def _():
        pltpu.sync_copy(part_sh.at[0, pl.ds(SW * si, SW)], sum_v)
        for k in range(1, NS):
            pltpu.sync_copy(part_sh.at[k, pl.ds(SW * si, SW)], tmp_v)

            def addv(i, _):
                sl = pl.ds(16 * i, 16)
                sum_v[sl] = sum_v[sl] + tmp_v[sl]
                return 0

            lax.fori_loop(0, SW // 16, addv, 0)
        pltpu.sync_copy(sum_v, out_hbm.at[ci, pl.ds(SW * si, SW)])


@functools.partial(
    pl.kernel,
    out_type=jax.ShapeDtypeStruct((NC, N, C), jnp.float32),
    mesh=_mesh,
    scratch_types=[
        pltpu.VMEM((CH // 2, K), jnp.int32),
        pltpu.VMEM((CH // 2, K), jnp.int32),
        pltpu.VMEM((K, C), jnp.float32),
        pltpu.VMEM((K, C), jnp.float32),
        pltpu.VMEM_SHARED((N, C), jnp.float32),
        pltpu.SemaphoreType.DMA,
        pltpu.SemaphoreType.DMA,
    ],
)
def _agg_kernel(z_hbm, src_hbm, dst_hbm, out_hbm, src_v, dst_v, rows0_v,
                rows1_v, acc_sh, sem_g, sem_s):
    ci = lax.axis_index("c")
    si = lax.axis_index("s")

    def zrow(i, _):
        for j in range(C // 16):
            rows0_v[i, pl.ds(16 * j, 16)] = jnp.zeros((16,), jnp.float32)
        return 0

    lax.fori_loop(0, K, zrow, 0)

    @pl.when(si < ND)
    def _():
        nfull, rem = divmod(DR, K)
        for t in range(nfull):
            pltpu.sync_copy(rows0_v, acc_sh.at[pl.ds(si * DR + t * K, K)])
        if rem:
            pltpu.sync_copy(rows0_v.at[pl.ds(0, rem)],
                            acc_sh.at[pl.ds(si * DR + nfull * K, rem)])

    plsc.subcore_barrier()

    HC = CH // 2
    for h in range(2):
        pltpu.sync_copy(src_hbm.at[ci, si, h], src_v)
        pltpu.sync_copy(dst_hbm.at[ci, si, h], dst_v)
        pltpu.async_copy(z_hbm.at[src_v.at[0]], rows0_v, sem_g)

        def body(g, _):
            for b in range(2):
                j = 2 * g + b
                cur = rows0_v if b == 0 else rows1_v
                oth = rows1_v if b == 0 else rows0_v
                pltpu.make_async_copy(z_hbm.at[src_v.at[j]], cur,
                                      sem_g).wait()
                pltpu.async_copy(cur, acc_sh.at[dst_v.at[j]], sem_s,
                                 add=True)

                @pl.when(j > 0)
                def _():
                    pltpu.make_async_copy(
                        oth, acc_sh.at[dst_v.at[j - 1]], sem_s).wait()

                @pl.when(j + 1 < HC)
                def _():
                    pltpu.async_copy(z_hbm.at[src_v.at[j + 1]], oth, sem_g)
            return 0

        lax.fori_loop(0, HC // 2, body, 0)
        pltpu.make_async_copy(rows1_v, acc_sh.at[dst_v.at[HC - 1]],
                              sem_s).wait()
    plsc.subcore_barrier()

    @pl.when(si < ND)
    def _():
        pltpu.sync_copy(acc_sh.at[pl.ds(si * DR, DR)],
                        out_hbm.at[ci, pl.ds(si * DR, DR)])


R = 1024
GRID = (NP // R,)


def _dinv_block(deg):
    return lax.rsqrt(deg[0] + deg[1] + 1.0)[:, None]


def _mm_prescale_body(x, w, deg, o):
    z = jnp.dot(x[:], w[:], preferred_element_type=jnp.float32)
    o[:] = z * _dinv_block(deg)


def _layer2_body(a, z1p, b1, deg, w, o):
    dinv = _dinv_block(deg)
    h = jnp.maximum(dinv * (a[0] + a[1] + z1p[:]) + b1[:], 0.0)
    z2 = jnp.dot(h, w[:], preferred_element_type=jnp.float32)
    o[:] = z2 * dinv


def _final_body(a, z2p, bmu, bls, deg, mu, ls):
    dinv = _dinv_block(deg)
    s = a[0] + a[1] + z2p[:]
    mu[:] = dinv * s[:, :C // 2] + bmu[:]
    ls[:] = dinv * s[:, C // 2:] + bls[:]


def _rows(width):
    return pl.BlockSpec((R, width), lambda i: (i, 0))


def _pair(width):
    return pl.BlockSpec((NC, R, width), lambda i: (0, i, 0))


def _degs():
    return pl.BlockSpec((NC, R), lambda i: (0, i))


def _full(shape):
    return pl.BlockSpec(shape, lambda i: tuple(0 for _ in shape))


def _nxc():
    return jax.ShapeDtypeStruct((N, C), jnp.float32)


def kernel(x, edge_index, W1, b1, W_mu, b_mu, W_ls, b_ls):
    src_r = edge_index[0].reshape(NC, NS, 2, CH // 2, K)
    dst_r = edge_index[1].reshape(NC, NS, 2, CH // 2, K)
    dst_flat = edge_index[1].reshape(NC, NS, EW)

    deg = _deg_kernel(dst_flat)

    z1p = pl.pallas_call(
        _mm_prescale_body,
        grid=GRID,
        in_specs=[_rows(C), _full((C, C)), _degs()],
        out_specs=_rows(C),
        out_shape=_nxc(),
    )(x, W1, deg)

    acc1 = _agg_kernel(z1p, src_r, dst_r)

    w_cat = jnp.concatenate([W_mu, W_ls], axis=1)

    z2p = pl.pallas_call(
        _layer2_body,
        grid=GRID,
        in_specs=[_pair(C), _rows(C), _full((1, C)), _degs(),
                  _full((C, C))],
        out_specs=_rows(C),
        out_shape=_nxc(),
    )(acc1, z1p, b1.reshape(1, C), deg, w_cat)

    acc2 = _agg_kernel(z2p, src_r, dst_r)

    mu, ls = pl.pallas_call(
        _final_body,
        grid=GRID,
        in_specs=[_pair(C), _rows(C), _full((1, C // 2)),
                  _full((1, C // 2)), _degs()],
        out_specs=[_rows(C // 2), _rows(C // 2)],
        out_shape=[jax.ShapeDtypeStruct((N, C // 2), jnp.float32),
                   jax.ShapeDtypeStruct((N, C // 2), jnp.float32)],
    )(acc2, z2p, b_mu.reshape(1, C // 2), b_ls.reshape(1, C // 2), deg)

    return (mu, ls)


# speedup vs baseline: 34.7831x; 1.0138x over previous
import functools

import jax
import jax.numpy as jnp
from jax import lax
from jax.experimental import pallas as pl
from jax.experimental.pallas import tpu as pltpu
from jax.experimental.pallas import tpu_sc as plsc

N = 10000
E = 320000
C = 128
NC = 2
NS = 16
K = 50
CH = E // (NC * NS * K)
GP = 5
EW = E // (NC * NS)
DR = 1000
ND = N // DR
ZC = 40
NP = 10240
SW = NP // ND

_mesh = plsc.VectorSubcoreMesh(
    core_axis_name="c", subcore_axis_name="s", num_cores=NC, num_subcores=NS)


@functools.partial(
    pl.kernel,
    out_type=jax.ShapeDtypeStruct((NC, NP), jnp.float32),
    mesh=_mesh,
    scratch_types=[
        pltpu.VMEM((EW,), jnp.int32),
        pltpu.VMEM((NP,), jnp.float32),
        pltpu.VMEM((SW,), jnp.float32),
        pltpu.VMEM((SW,), jnp.float32),
        pltpu.VMEM_SHARED((NS, NP), jnp.float32),
        pltpu.SemaphoreType.DMA,
    ],
    compiler_params=pltpu.CompilerParams(needs_layout_passes=False),
)
def _deg_kernel(dst_hbm, out_hbm, dst_v, cnt_v, tmp_v, sum_v, part_sh, sem):
    ci = lax.axis_index("c")
    si = lax.axis_index("s")
    zeros16 = jnp.zeros((16,), jnp.float32)
    ones16 = jnp.ones((16,), jnp.float32)

    def z1(i, _):
        cnt_v[pl.ds(16 * i, 16)] = zeros16
        return 0

    lax.fori_loop(0, NP // 16, z1, 0)

    pltpu.sync_copy(dst_hbm.at[ci, si], dst_v)

    def cnt(t, _):
        d = dst_v[pl.ds(16 * t, 16)]
        plsc.addupdate_scatter(cnt_v, [d], ones16)
        return 0

    lax.fori_loop(0, EW // 16, cnt, 0)

    pltpu.sync_copy(cnt_v, part_sh.at[si])
    plsc.subcore_barrier()

    @pl.when(si < ND)
    def _():
        pltpu.sync_copy(part_sh.at[0, pl.ds(SW * si, SW)], sum_v)
        for k in range(1, NS):
            pltpu.sync_copy(part_sh.at[k, pl.ds(SW * si, SW)], tmp_v)

            def addv(i, _):
                sl = pl.ds(16 * i, 16)
                sum_v[sl] = sum_v[sl] + tmp_v[sl]
                return 0

            lax.fori_loop(0, SW // 16, addv, 0)
        pltpu.sync_copy(sum_v, out_hbm.at[ci, pl.ds(SW * si, SW)])


@functools.partial(
    pl.kernel,
    out_type=jax.ShapeDtypeStruct((NC, N, C), jnp.float32),
    mesh=_mesh,
    scratch_types=[
        pltpu.VMEM((CH // GP, K), jnp.int32),
        pltpu.VMEM((CH // GP, K), jnp.int32),
        pltpu.VMEM((K, C), jnp.float32),
        pltpu.VMEM((K, C), jnp.float32),
        pltpu.VMEM((K, C), jnp.float32),
        pltpu.VMEM((K, C), jnp.float32),
        pltpu.VMEM_SHARED((N, C), jnp.float32),
        pltpu.SemaphoreType.DMA,
        pltpu.SemaphoreType.DMA,
    ],
)
def _agg_kernel(z_hbm, src_hbm, dst_hbm, out_hbm, src_v, dst_v, rows0_v,
                rows1_v, rows2_v, rows3_v, acc_sh, sem_g, sem_s):
    ci = lax.axis_index("c")
    si = lax.axis_index("s")
    bufs = (rows0_v, rows1_v, rows2_v, rows3_v)

    def zrow(i, _):
        for j in range(C // 16):
            rows0_v[i, pl.ds(16 * j, 16)] = jnp.zeros((16,), jnp.float32)
        return 0

    lax.fori_loop(0, K, zrow, 0)

    @pl.when(si < ND)
    def _():
        for t in range(DR // ZC):
            pltpu.sync_copy(rows0_v.at[pl.ds(0, ZC)],
                            acc_sh.at[pl.ds(si * DR + t * ZC, ZC)])

    plsc.subcore_barrier()

    HC = CH // GP
    for h in range(GP):
        pltpu.sync_copy(src_hbm.at[ci, si, h], src_v)
        pltpu.sync_copy(dst_hbm.at[ci, si, h], dst_v)
        pltpu.async_copy(z_hbm.at[src_v.at[0]], rows0_v, sem_g)
        pltpu.async_copy(z_hbm.at[src_v.at[1]], rows1_v, sem_g)

        def body(g, _):
            for b in range(4):
                j = 4 * g + b
                pltpu.make_async_copy(z_hbm.at[src_v.at[j]], bufs[b],
                                      sem_g).wait()
                pltpu.async_copy(bufs[b], acc_sh.at[dst_v.at[j]], sem_s,
                                 add=True)

                @pl.when(j >= 2)
                def _():
                    pltpu.make_async_copy(
                        bufs[(b - 2) % 4],
                        acc_sh.at[dst_v.at[j - 2]], sem_s).wait()

                @pl.when(j + 2 < HC)
                def _():
                    pltpu.async_copy(z_hbm.at[src_v.at[j + 2]],
                                     bufs[(b + 2) % 4], sem_g)
            return 0

        lax.fori_loop(0, HC // 4, body, 0)
        pltpu.make_async_copy(bufs[(HC - 2) % 4],
                              acc_sh.at[dst_v.at[HC - 2]], sem_s).wait()
        pltpu.make_async_copy(bufs[(HC - 1) % 4],
                              acc_sh.at[dst_v.at[HC - 1]], sem_s).wait()
    plsc.subcore_barrier()

    @pl.when(si < ND)
    def _():
        pltpu.sync_copy(acc_sh.at[pl.ds(si * DR, DR)],
                        out_hbm.at[ci, pl.ds(si * DR, DR)])


R = 1024
GRID = (NP // R,)


def _dinv_block(deg):
    return lax.rsqrt(deg[0] + deg[1] + 1.0)[:, None]


def _mm_prescale_body(x, w, deg, o):
    z = jnp.dot(x[:], w[:], preferred_element_type=jnp.float32)
    o[:] = z * _dinv_block(deg)


def _layer2_body(a, z1p, b1, deg, w, o):
    dinv = _dinv_block(deg)
    h = jnp.maximum(dinv * (a[0] + a[1] + z1p[:]) + b1[:], 0.0)
    z2 = jnp.dot(h, w[:], preferred_element_type=jnp.float32)
    o[:] = z2 * dinv


def _final_body(a, z2p, bmu, bls, deg, mu, ls):
    dinv = _dinv_block(deg)
    s = a[0] + a[1] + z2p[:]
    mu[:] = dinv * s[:, :C // 2] + bmu[:]
    ls[:] = dinv * s[:, C // 2:] + bls[:]


def _rows(width):
    return pl.BlockSpec((R, width), lambda i: (i, 0))


def _pair(width):
    return pl.BlockSpec((NC, R, width), lambda i: (0, i, 0))


def _degs():
    return pl.BlockSpec((NC, R), lambda i: (0, i))


def _full(shape):
    return pl.BlockSpec(shape, lambda i: tuple(0 for _ in shape))


def _nxc():
    return jax.ShapeDtypeStruct((N, C), jnp.float32)


def kernel(x, edge_index, W1, b1, W_mu, b_mu, W_ls, b_ls):
    src_r = edge_index[0].reshape(NC, NS, GP, CH // GP, K)
    dst_r = edge_index[1].reshape(NC, NS, GP, CH // GP, K)
    dst_flat = edge_index[1].reshape(NC, NS, EW)

    deg = _deg_kernel(dst_flat)

    z1p = pl.pallas_call(
        _mm_prescale_body,
        grid=GRID,
        in_specs=[_rows(C), _full((C, C)), _degs()],
        out_specs=_rows(C),
        out_shape=_nxc(),
    )(x, W1, deg)

    acc1 = _agg_kernel(z1p, src_r, dst_r)

    w_cat = jnp.concatenate([W_mu, W_ls], axis=1)

    z2p = pl.pallas_call(
        _layer2_body,
        grid=GRID,
        in_specs=[_pair(C), _rows(C), _full((1, C)), _degs(),
                  _full((C, C))],
        out_specs=_rows(C),
        out_shape=_nxc(),
    )(acc1, z1p, b1.reshape(1, C), deg, w_cat)

    acc2 = _agg_kernel(z2p, src_r, dst_r)

    mu, ls = pl.pallas_call(
        _final_body,
        grid=GRID,
        in_specs=[_pair(C), _rows(C), _full((1, C // 2)),
                  _full((1, C // 2)), _degs()],
        out_specs=[_rows(C // 2), _rows(C // 2)],
        out_shape=[jax.ShapeDtypeStruct((N, C // 2), jnp.float32),
                   jax.ShapeDtypeStruct((N, C // 2), jnp.float32)],
    )(acc2, z2p, b_mu.reshape(1, C // 2), b_ls.reshape(1, C // 2), deg)

    return (mu, ls)
